# baseline (speedup 1.0000x reference)
import functools

import jax
import jax.numpy as jnp
from jax import lax
from jax.experimental import pallas as pl
from jax.experimental.pallas import tpu as pltpu
from jax.experimental.pallas import tpu_sc as plsc

_S = 26
_NB = 4096
_D = 128
_NC = 2
_NS = 16
_NW = _NC * _NS
_CPW = _NB // _NW
_CH = 128
_HB = _CPW // _CH
_NCHUNK = _S * _HB
_NBUF = 2
_NGRP = _NCHUNK // _NBUF


def _make_gather():
  mesh = plsc.VectorSubcoreMesh(core_axis_name="c", subcore_axis_name="s")

  @functools.partial(
      pl.kernel,
      mesh=mesh,
      out_type=jax.ShapeDtypeStruct((_S, _NB, _D), jnp.float32),
      scratch_types=[
          pltpu.VMEM((_S, _CPW), jnp.int32),
          pltpu.VMEM((_NBUF, _CH, _D), jnp.float32),
      ] + [pltpu.SemaphoreType.DMA] * (2 * _NBUF),
  )
  def k(idx_hbm, table_hbm, out_hbm, idx_v, rows_v, *sems):
    gsem = sems[:_NBUF]
    osem = sems[_NBUF:]
    wid = lax.axis_index("s") * _NC + lax.axis_index("c")
    cbase = wid * _CPW
    pltpu.sync_copy(idx_hbm.at[:, pl.ds(cbase, _CPW)], idx_v)

    def gather_desc(b, j):
      s = j // _HB
      h = j % _HB
      return pltpu.make_async_copy(
          table_hbm.at[idx_v.at[s, pl.ds(h * _CH, _CH)]],
          rows_v.at[b], gsem[b])

    def store_desc(b, j):
      s = j // _HB
      h = j % _HB
      return pltpu.make_async_copy(
          rows_v.at[b], out_hbm.at[s, pl.ds(cbase + h * _CH, _CH)], osem[b])

    def group(g, carry):
      for b in range(_NBUF):
        j = g * _NBUF + b

        @pl.when(g > 0)
        def _(b=b, j=j):
          store_desc(b, j - _NBUF).wait()

        gather_desc(b, j).start()
      for b in range(_NBUF):
        j = g * _NBUF + b
        gather_desc(b, j).wait()
        store_desc(b, j).start()
      return carry

    lax.fori_loop(0, _NGRP, group, 0)
    for b in range(_NBUF):
      store_desc(b, _NCHUNK - _NBUF + b).wait()

  return k


_gather = _make_gather()


def kernel(x, weight):
  out_t = _gather(x.T.astype(jnp.int32), weight)
  return out_t.transpose(1, 0, 2)


# speedup vs baseline: 1.0966x; 1.0966x over previous
import functools

import jax
import jax.numpy as jnp
from jax import lax
from jax.experimental import pallas as pl
from jax.experimental.pallas import tpu as pltpu
from jax.experimental.pallas import tpu_sc as plsc

_S = 26
_NB = 4096
_D = 128
_NC = 2
_NS = 16
_NW = _NC * _NS
_CPW = _NB // _NW
_CH = 32
_HB = _CPW // _CH
_NCHUNK = _S * _HB
_GRP = 13
_NG = _NCHUNK // _GRP


def _make_gather():
  mesh = plsc.VectorSubcoreMesh(core_axis_name="c", subcore_axis_name="s")

  @functools.partial(
      pl.kernel,
      mesh=mesh,
      out_type=jax.ShapeDtypeStruct((_S, _NB, _D), jnp.float32),
      scratch_types=[
          pltpu.VMEM((_S, _CPW), jnp.int32),
          pltpu.VMEM((2 * _GRP, _CH, _D), jnp.float32),
          pltpu.SemaphoreType.DMA,
          pltpu.SemaphoreType.DMA,
      ],
  )
  def k(idx_hbm, table_hbm, out_hbm, idx_v, rows_v, gsem, osem):
    wid = lax.axis_index("s") * _NC + lax.axis_index("c")
    cbase = wid * _CPW
    pltpu.sync_copy(idx_hbm.at[:, pl.ds(cbase, _CPW)], idx_v)

    def gather_desc(slot, j):
      s = j // _HB
      h = j % _HB
      return pltpu.make_async_copy(
          table_hbm.at[idx_v.at[s, pl.ds(h * _CH, _CH)]],
          rows_v.at[slot], gsem)

    def store_desc(slot, j):
      s = j // _HB
      h = j % _HB
      return pltpu.make_async_copy(
          rows_v.at[slot], out_hbm.at[s, pl.ds(cbase + h * _CH, _CH)], osem)

    def fire_gathers(k_grp, half):
      for i in range(_GRP):
        gather_desc(half * _GRP + i, k_grp * _GRP + i).start()

    def wait_gathers(k_grp, half):
      for i in range(_GRP):
        gather_desc(half * _GRP + i, k_grp * _GRP + i).wait()

    def fire_stores(k_grp, half):
      for i in range(_GRP):
        store_desc(half * _GRP + i, k_grp * _GRP + i).start()

    def wait_stores(k_grp, half):
      for i in range(_GRP):
        store_desc(half * _GRP + i, k_grp * _GRP + i).wait()

    fire_gathers(0, 0)

    def pair(p, carry):
      k0 = 2 * p

      @pl.when(p > 0)
      def _():
        wait_stores(k0 - 1, 1)

      fire_gathers(k0 + 1, 1)
      wait_gathers(k0, 0)
      fire_stores(k0, 0)
      k1 = k0 + 1
      wait_stores(k1 - 1, 0)

      @pl.when(p < _NG // 2 - 1)
      def _():
        fire_gathers(k1 + 1, 0)

      wait_gathers(k1, 1)
      fire_stores(k1, 1)
      return carry

    lax.fori_loop(0, _NG // 2, pair, 0)
    wait_stores(_NG - 1, 1)

  return k


_gather = _make_gather()


def kernel(x, weight):
  out_t = _gather(x.T.astype(jnp.int32), weight)
  return out_t.transpose(1, 0, 2)
